# initial kernel scaffold (unmeasured)
import jax
import jax.numpy as jnp
from jax import lax
from jax.experimental import pallas as pl
from jax.experimental.pallas import tpu as pltpu

N_DEV = 32
M = 4096
N = 2048
CHUNK = M // N_DEV


def kernel(x, w_mat, scale_x, scale_w):
    m, k_per = x.shape
    k_per2, n = w_mat.shape
    assert m == M and n == N and k_per == k_per2

    s = (scale_x * scale_w).reshape(1, 1)

    def body(x_ref, w_ref, s_ref, out_ref, comm_ref, send_sems, recv_sems):
        my = lax.axis_index("i")
        left = jnp.remainder(my - 1, N_DEV)
        right = jnp.remainder(my + 1, N_DEV)

        out_ref[...] = jnp.dot(
            x_ref[...], w_ref[...], preferred_element_type=jnp.float32
        )

        barrier_sem = pltpu.get_barrier_semaphore()
        for nbr in (left, right):
            pl.semaphore_signal(
                barrier_sem, inc=1,
                device_id=(nbr,), device_id_type=pl.DeviceIdType.MESH,
            )
        pl.semaphore_wait(barrier_sem, 2)

        for t in range(N_DEV - 1):
            s_idx = jnp.remainder(my - t, N_DEV)
            r_idx = jnp.remainder(my - t - 1, N_DEV)
            slot = (t + 1) % 2
            rdma = pltpu.make_async_remote_copy(
                src_ref=out_ref.at[pl.ds(s_idx * CHUNK, CHUNK), :],
                dst_ref=comm_ref.at[slot],
                send_sem=send_sems.at[t % 2],
                recv_sem=recv_sems.at[slot],
                device_id=(right,),
                device_id_type=pl.DeviceIdType.MESH,
            )
            rdma.start()
            rdma.wait()
            out_ref[pl.ds(r_idx * CHUNK, CHUNK), :] = (
                out_ref[pl.ds(r_idx * CHUNK, CHUNK), :] + comm_ref[slot]
            )

        g = jnp.remainder(my + 1, N_DEV)
        y = out_ref[pl.ds(g * CHUNK, CHUNK), :] * s_ref[0, 0]
        z = y / (1.0 + jnp.exp(-jnp.clip(y, -60.0, 60.0)))
        comm_ref[1] = z
        out_ref[pl.ds(g * CHUNK, CHUNK), :] = z

        for t in range(N_DEV - 1, 2 * (N_DEV - 1)):
            h2 = t - (N_DEV - 1)
            send_slot = t % 2
            recv_slot = (t + 1) % 2
            rdma = pltpu.make_async_remote_copy(
                src_ref=comm_ref.at[send_slot],
                dst_ref=comm_ref.at[recv_slot],
                send_sem=send_sems.at[send_slot],
                recv_sem=recv_sems.at[recv_slot],
                device_id=(right,),
                device_id_type=pl.DeviceIdType.MESH,
            )
            rdma.start()
            rdma.wait()
            c = jnp.remainder(my - h2, N_DEV)
            out_ref[pl.ds(c * CHUNK, CHUNK), :] = comm_ref[recv_slot]

    return pl.pallas_call(
        body,
        out_shape=jax.ShapeDtypeStruct((M, N), jnp.float32),
        in_specs=[
            pl.BlockSpec(memory_space=pltpu.VMEM),
            pl.BlockSpec(memory_space=pltpu.VMEM),
            pl.BlockSpec(memory_space=pltpu.SMEM),
        ],
        out_specs=pl.BlockSpec(memory_space=pltpu.VMEM),
        scratch_shapes=[
            pltpu.VMEM((2, CHUNK, N), jnp.float32),
            pltpu.SemaphoreType.DMA((2,)),
            pltpu.SemaphoreType.DMA((2,)),
        ],
        compiler_params=pltpu.CompilerParams(collective_id=0),
    )(x, w_mat, s)


# baseline (device time: 862004 ns/iter reference)
import jax
import jax.numpy as jnp
from jax import lax
from jax.experimental import pallas as pl
from jax.experimental.pallas import tpu as pltpu

N_DEV = 32
M = 4096
N = 2048
CHUNK = M // N_DEV


def kernel(x, w_mat, scale_x, scale_w):
    m, k_per = x.shape
    k_per2, n = w_mat.shape
    assert m == M and n == N and k_per == k_per2

    s = (scale_x * scale_w).reshape(1, 1)

    def body(x_ref, w_ref, s_ref, out_ref, comm_ref, send_sems, recv_sems):
        my = lax.axis_index("i")
        left = jnp.remainder(my - 1, N_DEV)
        right = jnp.remainder(my + 1, N_DEV)

        out_ref[...] = jnp.dot(
            x_ref[...], w_ref[...], preferred_element_type=jnp.float32
        )

        barrier_sem = pltpu.get_barrier_semaphore()
        for nbr in (left, right):
            pl.semaphore_signal(
                barrier_sem, inc=1,
                device_id=(nbr,), device_id_type=pl.DeviceIdType.MESH,
            )
        pl.semaphore_wait(barrier_sem, 2)

        for t in range(N_DEV - 1):
            s_idx = jnp.remainder(my - t, N_DEV)
            r_idx = jnp.remainder(my - t - 1, N_DEV)
            slot = (t + 1) % 2
            rdma = pltpu.make_async_remote_copy(
                src_ref=out_ref.at[pl.ds(s_idx * CHUNK, CHUNK), :],
                dst_ref=comm_ref.at[slot],
                send_sem=send_sems.at[t % 2],
                recv_sem=recv_sems.at[slot],
                device_id=(right,),
                device_id_type=pl.DeviceIdType.MESH,
            )
            rdma.start()
            rdma.wait()
            out_ref[pl.ds(r_idx * CHUNK, CHUNK), :] = (
                out_ref[pl.ds(r_idx * CHUNK, CHUNK), :] + comm_ref[slot]
            )

        g = jnp.remainder(my + 1, N_DEV)
        y = out_ref[pl.ds(g * CHUNK, CHUNK), :] * s_ref[0, 0]
        z = y / (1.0 + jnp.exp(-jnp.clip(y, -60.0, 60.0)))
        comm_ref[1] = z
        out_ref[pl.ds(g * CHUNK, CHUNK), :] = z

        for t in range(N_DEV - 1, 2 * (N_DEV - 1)):
            h2 = t - (N_DEV - 1)
            send_slot = t % 2
            recv_slot = (t + 1) % 2
            rdma = pltpu.make_async_remote_copy(
                src_ref=comm_ref.at[send_slot],
                dst_ref=comm_ref.at[recv_slot],
                send_sem=send_sems.at[send_slot],
                recv_sem=recv_sems.at[recv_slot],
                device_id=(right,),
                device_id_type=pl.DeviceIdType.MESH,
            )
            rdma.start()
            rdma.wait()
            c = jnp.remainder(my - h2, N_DEV)
            out_ref[pl.ds(c * CHUNK, CHUNK), :] = comm_ref[recv_slot]

    return pl.pallas_call(
        body,
        out_shape=jax.ShapeDtypeStruct((M, N), jnp.float32),
        in_specs=[
            pl.BlockSpec(memory_space=pltpu.VMEM),
            pl.BlockSpec(memory_space=pltpu.VMEM),
            pl.BlockSpec(memory_space=pltpu.SMEM),
        ],
        out_specs=pl.BlockSpec(memory_space=pltpu.VMEM),
        scratch_shapes=[
            pltpu.VMEM((2, CHUNK, N), jnp.float32),
            pltpu.SemaphoreType.DMA((2,)),
            pltpu.SemaphoreType.DMA((2,)),
        ],
        compiler_params=pltpu.CompilerParams(
            collective_id=0, vmem_limit_bytes=100 * 1024 * 1024
        ),
    )(x, w_mat, s)
